# baseline (speedup 1.0000x reference)
import jax
import jax.numpy as jnp
from jax import lax
from jax.experimental import pallas as pl
from jax.experimental.pallas import tpu as pltpu
from jax.experimental.pallas import tpu_sc as plsc

D = 64
N_W = 50000
N_WP = 51200
E = 400000
NC = 2
NS = 16
NW = NC * NS
CT = 640
CW = 512
H = CW // 2
E_CHUNKS = E // CT
W_CHUNKS = N_WP // CW
RB = 8

_GDN = lax.GatherDimensionNumbers(
    offset_dims=(), collapsed_slice_dims=(0,), start_index_map=(0,))


def _vgather(src, idx):
    return lax.gather(src, idx[:, None], _GDN, (1,),
                      mode=lax.GatherScatterMode.PROMISE_IN_BOUNDS)


def _body(tffrac_ws, tffrac_we, widp, tf_T, emb2,
          w_outT, ws_outT, we_outT, tf_v, sem):
    w = lax.axis_index("s") * NC + lax.axis_index("c")

    pltpu.sync_copy(tf_T, tf_v)
    iota16 = lax.iota(jnp.int32, 16)
    mask10 = iota16 < 10

    def wid_phase(idxw_v, idx2_v, rows2_v, wbuf):
        iters = (W_CHUNKS + NW - 1) // NW

        def step(t, carry):
            chunk = w + t * NW

            @pl.when(chunk < W_CHUNKS)
            def _():
                base = chunk * CW
                pltpu.sync_copy(widp.at[pl.ds(base, CW)], idxw_v)
                for h in range(2):
                    def halve(g, c2):
                        v = idxw_v[pl.ds(h * H + g * 16, 16)]
                        idx2_v[pl.ds(g * 16, 16)] = v >> 1
                        return c2

                    lax.fori_loop(0, H // 16, halve, 0)
                    pltpu.async_copy(emb2.at[idx2_v], rows2_v, sem).wait()

                    def select(g, c2):
                        rowv = iota16 + g * 16
                        par = idxw_v[pl.ds(h * H + g * 16, 16)] & 1
                        colbase = par * D
                        for c in range(D):
                            vals = plsc.load_gather(
                                rows2_v, [rowv, colbase + c])
                            wbuf[c, pl.ds(h * H + g * 16, 16)] = vals
                        return c2

                    lax.fori_loop(0, H // 16, select, 0)
                pltpu.sync_copy(wbuf, w_outT.at[:, pl.ds(base, CW)])

            return carry

        lax.fori_loop(0, iters, step, 0)

    def tf_phase(idxs, bufs, sem_i, sem_s):
        def expand(idx_v, buf):
            for rb in range(0, D, RB):
                trows = [
                    plsc.load_gather(
                        tf_v, [jnp.full((16,), r, jnp.int32), iota16],
                        mask=mask10)
                    for r in range(rb, rb + RB)
                ]

                def gblock(g, c2):
                    idxg = idx_v[pl.ds(g * 16, 16)]
                    for j in range(RB):
                        buf[rb + j, pl.ds(g * 16, 16)] = _vgather(
                            trows[j], idxg)
                    return c2

                lax.fori_loop(0, CT // 16, gblock, 0)

        def job(idx_hbm, outT):
            iters = (E_CHUNKS + NW - 1) // NW

            for s in range(2):
                ch = w + s * NW

                @pl.when(ch < E_CHUNKS)
                def _():
                    pltpu.async_copy(idx_hbm.at[pl.ds(ch * CT, CT)],
                                     idxs[s], sem_i[s])

            def pair(u, carry):
                for s in range(2):
                    t = 2 * u + s
                    ch = w + t * NW

                    @pl.when(ch < E_CHUNKS)
                    def _():
                        pltpu.make_async_copy(
                            idx_hbm.at[pl.ds(0, CT)], idxs[s],
                            sem_i[s]).wait()

                        @pl.when(t >= 2)
                        def _():
                            pltpu.make_async_copy(
                                bufs[s], outT.at[:, pl.ds(0, CT)],
                                sem_s[s]).wait()

                        expand(idxs[s], bufs[s])

                        ch2 = w + (t + 2) * NW

                        @pl.when(ch2 < E_CHUNKS)
                        def _():
                            pltpu.async_copy(
                                idx_hbm.at[pl.ds(ch2 * CT, CT)],
                                idxs[s], sem_i[s])

                        pltpu.async_copy(bufs[s],
                                         outT.at[:, pl.ds(ch * CT, CT)],
                                         sem_s[s])

                return carry

            lax.fori_loop(0, (iters + 1) // 2, pair, 0)
            for s in range(2):
                pltpu.make_async_copy(bufs[s], outT.at[:, pl.ds(0, CT)],
                                      sem_s[s]).wait()

        job(tffrac_ws, ws_outT)
        job(tffrac_we, we_outT)

    pl.run_scoped(
        wid_phase,
        pltpu.VMEM((CW,), jnp.int32),
        pltpu.VMEM((H,), jnp.int32),
        pltpu.VMEM((H, 2 * D), jnp.float32),
        pltpu.VMEM((D, CW), jnp.float32),
    )
    pl.run_scoped(
        tf_phase,
        [pltpu.VMEM((CT,), jnp.int32)] * 2,
        [pltpu.VMEM((D, CT), jnp.float32)] * 2,
        [pltpu.SemaphoreType.DMA] * 2,
        [pltpu.SemaphoreType.DMA] * 2,
    )


TB = 256
T_CHUNKS = (1000000 - 64) // TB


def _tbody(emb_T, tail_T, emb2, bins, bouts, bin_t, sem_i, sem_s):
    w = lax.axis_index("s") * NC + lax.axis_index("c")
    iota16 = lax.iota(jnp.int32, 16)

    def tcompute(bin_, bout, npair):
        def jloop(j, c2):
            for q in range(2):
                colv = jnp.full((16,), 0, jnp.int32) + (2 * j + q)
                for k2 in range(4):
                    vals = plsc.load_gather(bin_, [iota16 + k2 * 16, colv])
                    bout[j, pl.ds(q * D + k2 * 16, 16)] = vals
            return c2

        lax.fori_loop(0, npair, jloop, 0)

    for s in range(2):
        ch = w + s * NW

        @pl.when(ch < T_CHUNKS)
        def _():
            pltpu.async_copy(emb_T.at[:, pl.ds(ch * TB, TB)], bins[s],
                             sem_i[s])

    iters = (T_CHUNKS + NW - 1) // NW

    def pair(u, carry):
        for s in range(2):
            t = 2 * u + s
            ch = w + t * NW

            @pl.when(ch < T_CHUNKS)
            def _():
                pltpu.make_async_copy(emb_T.at[:, pl.ds(0, TB)], bins[s],
                                      sem_i[s]).wait()

                @pl.when(t >= 2)
                def _():
                    pltpu.make_async_copy(bouts[s],
                                          emb2.at[pl.ds(0, TB // 2)],
                                          sem_s[s]).wait()

                tcompute(bins[s], bouts[s], TB // 2)

                ch2 = w + (t + 2) * NW

                @pl.when(ch2 < T_CHUNKS)
                def _():
                    pltpu.async_copy(emb_T.at[:, pl.ds(ch2 * TB, TB)],
                                     bins[s], sem_i[s])

                pltpu.async_copy(bouts[s],
                                 emb2.at[pl.ds(ch * (TB // 2), TB // 2)],
                                 sem_s[s])

        return carry

    lax.fori_loop(0, (iters + 1) // 2, pair, 0)
    for s in range(2):
        pltpu.make_async_copy(bouts[s], emb2.at[pl.ds(0, TB // 2)],
                              sem_s[s]).wait()

    @pl.when(w == NW - 1)
    def _():
        pltpu.sync_copy(tail_T, bin_t)
        tcompute(bin_t, bouts[0], 32)
        pltpu.sync_copy(bouts[0].at[pl.ds(0, 32)],
                        emb2.at[pl.ds(499968, 32)])


def kernel(tffrac_ws, tffrac_we, wid, TF_table, embed_table):
    mesh = plsc.VectorSubcoreMesh(core_axis_name="c", subcore_axis_name="s")
    tk = pl.kernel(
        _tbody,
        mesh=mesh,
        compiler_params=pltpu.CompilerParams(needs_layout_passes=False),
        out_type=jax.ShapeDtypeStruct((N_W * 10, 2 * D), jnp.float32),
        scratch_types=[
            [pltpu.VMEM((D, TB), jnp.float32)] * 2,
            [pltpu.VMEM((TB // 2, 2 * D), jnp.float32)] * 2,
            pltpu.VMEM((D, D), jnp.float32),
            [pltpu.SemaphoreType.DMA] * 2,
            [pltpu.SemaphoreType.DMA] * 2,
        ],
    )
    k = pl.kernel(
        _body,
        mesh=mesh,
        compiler_params=pltpu.CompilerParams(needs_layout_passes=False),
        out_type=(
            jax.ShapeDtypeStruct((D, N_WP), jnp.float32),
            jax.ShapeDtypeStruct((D, E), jnp.float32),
            jax.ShapeDtypeStruct((D, E), jnp.float32),
        ),
        scratch_types=[
            pltpu.VMEM((D, 10), jnp.float32),
            pltpu.SemaphoreType.DMA,
        ],
    )
    widp = jnp.concatenate(
        [wid.astype(jnp.int32), jnp.arange(N_WP - N_W, dtype=jnp.int32)])
    emb2 = tk(embed_table.T, embed_table[1000000 - 64:].T)
    w_T, ws_T, we_T = k(tffrac_ws.astype(jnp.int32),
                        tffrac_we.astype(jnp.int32),
                        widp, TF_table.T, emb2)
    return w_T[:, :N_W].T, ws_T.T, we_T.T


# speedup vs baseline: 2.1772x; 2.1772x over previous
import jax
import jax.numpy as jnp
from jax import lax
from jax.experimental import pallas as pl
from jax.experimental.pallas import tpu as pltpu
from jax.experimental.pallas import tpu_sc as plsc

D = 64
N_W = 50000
N_WP = 51200
E = 400000
NC = 2
NS = 16
NW = NC * NS
CT = 640
CW = 512
H = CW // 2
E_CHUNKS = E // CT
W_CHUNKS = N_WP // CW
RB = 8

_GDN = lax.GatherDimensionNumbers(
    offset_dims=(), collapsed_slice_dims=(0,), start_index_map=(0,))


def _vgather(src, idx):
    return lax.gather(src, idx[:, None], _GDN, (1,),
                      mode=lax.GatherScatterMode.PROMISE_IN_BOUNDS)


def _body(tffrac_ws, tffrac_we, widp, tf_T, emb2,
          w_outT, ws_outT, we_outT, tf_v, sem):
    w = lax.axis_index("s") * NC + lax.axis_index("c")

    pltpu.sync_copy(tf_T, tf_v)
    iota16 = lax.iota(jnp.int32, 16)
    mask10 = iota16 < 10

    def wid_phase(idxw_v, idx2_v, rows2_v, wbuf):
        iters = (W_CHUNKS + NW - 1) // NW

        def step(t, carry):
            chunk = w + t * NW

            @pl.when(chunk < W_CHUNKS)
            def _():
                base = chunk * CW
                pltpu.sync_copy(widp.at[pl.ds(base, CW)], idxw_v)
                for h in range(2):
                    def halve(g, c2):
                        v = idxw_v[pl.ds(h * H + g * 16, 16)]
                        idx2_v[pl.ds(g * 16, 16)] = v >> 1
                        return c2

                    lax.fori_loop(0, H // 16, halve, 0)
                    pltpu.async_copy(emb2.at[idx2_v], rows2_v, sem).wait()

                    def select(g, c2):
                        rowv = iota16 + g * 16
                        par = idxw_v[pl.ds(h * H + g * 16, 16)] & 1
                        colbase = par * D
                        for c in range(D):
                            vals = plsc.load_gather(
                                rows2_v, [rowv, colbase + c])
                            wbuf[c, pl.ds(h * H + g * 16, 16)] = vals
                        return c2

                    lax.fori_loop(0, H // 16, select, 0)
                pltpu.sync_copy(wbuf, w_outT.at[:, pl.ds(base, CW)])

            return carry

        lax.fori_loop(0, iters, step, 0)

    def tf_phase(idxs, bufs, sem_i, sem_s):
        def expand(idx_v, buf):
            for rb in range(0, D, RB):
                trows = [
                    plsc.load_gather(
                        tf_v, [jnp.full((16,), r, jnp.int32), iota16],
                        mask=mask10)
                    for r in range(rb, rb + RB)
                ]

                def gblock(g, c2):
                    idxg = idx_v[pl.ds(g * 16, 16)]
                    for j in range(RB):
                        buf[rb + j, pl.ds(g * 16, 16)] = _vgather(
                            trows[j], idxg)
                    return c2

                lax.fori_loop(0, CT // 16, gblock, 0)

        def job(idx_hbm, outT):
            iters = (E_CHUNKS + NW - 1) // NW

            for s in range(2):
                ch = w + s * NW

                @pl.when(ch < E_CHUNKS)
                def _():
                    pltpu.async_copy(idx_hbm.at[pl.ds(ch * CT, CT)],
                                     idxs[s], sem_i[s])

            def pair(u, carry):
                for s in range(2):
                    t = 2 * u + s
                    ch = w + t * NW

                    @pl.when(ch < E_CHUNKS)
                    def _():
                        pltpu.make_async_copy(
                            idx_hbm.at[pl.ds(0, CT)], idxs[s],
                            sem_i[s]).wait()

                        @pl.when(t >= 2)
                        def _():
                            pltpu.make_async_copy(
                                bufs[s], outT.at[:, pl.ds(0, CT)],
                                sem_s[s]).wait()

                        expand(idxs[s], bufs[s])

                        ch2 = w + (t + 2) * NW

                        @pl.when(ch2 < E_CHUNKS)
                        def _():
                            pltpu.async_copy(
                                idx_hbm.at[pl.ds(ch2 * CT, CT)],
                                idxs[s], sem_i[s])

                        pltpu.async_copy(bufs[s],
                                         outT.at[:, pl.ds(ch * CT, CT)],
                                         sem_s[s])

                return carry

            lax.fori_loop(0, (iters + 1) // 2, pair, 0)
            for s in range(2):
                pltpu.make_async_copy(bufs[s], outT.at[:, pl.ds(0, CT)],
                                      sem_s[s]).wait()

        job(tffrac_ws, ws_outT)
        job(tffrac_we, we_outT)

    pl.run_scoped(
        wid_phase,
        pltpu.VMEM((CW,), jnp.int32),
        pltpu.VMEM((H,), jnp.int32),
        pltpu.VMEM((H, 2 * D), jnp.float32),
        pltpu.VMEM((D, CW), jnp.float32),
    )
    pl.run_scoped(
        tf_phase,
        [pltpu.VMEM((CT,), jnp.int32)] * 2,
        [pltpu.VMEM((D, CT), jnp.float32)] * 2,
        [pltpu.SemaphoreType.DMA] * 2,
        [pltpu.SemaphoreType.DMA] * 2,
    )


def kernel(tffrac_ws, tffrac_we, wid, TF_table, embed_table):
    mesh = plsc.VectorSubcoreMesh(core_axis_name="c", subcore_axis_name="s")
    k = pl.kernel(
        _body,
        mesh=mesh,
        compiler_params=pltpu.CompilerParams(needs_layout_passes=False),
        out_type=(
            jax.ShapeDtypeStruct((D, N_WP), jnp.float32),
            jax.ShapeDtypeStruct((D, E), jnp.float32),
            jax.ShapeDtypeStruct((D, E), jnp.float32),
        ),
        scratch_types=[
            pltpu.VMEM((D, 10), jnp.float32),
            pltpu.SemaphoreType.DMA,
        ],
    )
    widp = jnp.concatenate(
        [wid.astype(jnp.int32), jnp.arange(N_WP - N_W, dtype=jnp.int32)])
    emb2 = embed_table.reshape(N_W * 10, 2 * D)
    w_T, ws_T, we_T = k(tffrac_ws.astype(jnp.int32),
                        tffrac_we.astype(jnp.int32),
                        widp, TF_table.T, emb2)
    return w_T[:, :N_W].T, ws_T.T, we_T.T
